# baseline (speedup 1.0000x reference)
import functools

import jax
import jax.numpy as jnp
from jax import lax
from jax.experimental import pallas as pl
from jax.experimental.pallas import tpu as pltpu
from jax.experimental.pallas import tpu_sc as plsc

_NC = 2
_NS = 16
_NW = _NC * _NS
_CHUNK = 8192
_L = 16


def _sc_scatter(xt, srcp, dstp, probsp, n, d):
    e_pad = srcp.shape[0]
    chunks = e_pad // _CHUNK
    cpt = d // _NW
    flat = cpt * n
    groups = _CHUNK // _L

    mesh = plsc.VectorSubcoreMesh(core_axis_name="c", subcore_axis_name="s")

    @functools.partial(
        pl.kernel,
        out_type=jax.ShapeDtypeStruct((_NW, flat), jnp.float32),
        mesh=mesh,
        compiler_params=pltpu.CompilerParams(needs_layout_passes=False),
        scratch_types=[
            pltpu.VMEM((flat,), jnp.float32),
            pltpu.VMEM((flat,), jnp.float32),
            pltpu.VMEM((2, _CHUNK), jnp.int32),
            pltpu.VMEM((2, _CHUNK), jnp.int32),
            pltpu.VMEM((2, _CHUNK), jnp.float32),
            pltpu.SemaphoreType.DMA,
            pltpu.SemaphoreType.DMA,
        ],
    )
    def k(xt_hbm, src_hbm, dst_hbm, probs_hbm, out_hbm,
          xloc, accl, sbuf, dbuf, pbuf, gsem, isem):
        cid = lax.axis_index("c")
        sid = lax.axis_index("s")
        gid = cid * _NS + sid

        pltpu.async_copy(xt_hbm.at[gid], xloc, gsem)

        def zbody(i, carry):
            accl[pl.ds(i * _L, _L)] = jnp.zeros((_L,), jnp.float32)
            return carry
        lax.fori_loop(0, flat // _L, zbody, 0)

        pltpu.async_copy(src_hbm.at[pl.ds(0, _CHUNK)], sbuf.at[0], isem)
        pltpu.async_copy(dst_hbm.at[pl.ds(0, _CHUNK)], dbuf.at[0], isem)
        pltpu.async_copy(probs_hbm.at[pl.ds(0, _CHUNK)], pbuf.at[0], isem)
        pltpu.make_async_copy(xt_hbm.at[gid], xloc, gsem).wait()

        def chunk_body(ci, carry):
            b = ci % 2
            nb = 1 - b
            pltpu.make_async_copy(src_hbm.at[pl.ds(0, _CHUNK)],
                                  sbuf.at[0], isem).wait()
            pltpu.make_async_copy(dst_hbm.at[pl.ds(0, _CHUNK)],
                                  dbuf.at[0], isem).wait()
            pltpu.make_async_copy(probs_hbm.at[pl.ds(0, _CHUNK)],
                                  pbuf.at[0], isem).wait()
            nc = jnp.minimum(ci + 1, chunks - 1) * _CHUNK
            pltpu.async_copy(src_hbm.at[pl.ds(nc, _CHUNK)], sbuf.at[nb], isem)
            pltpu.async_copy(dst_hbm.at[pl.ds(nc, _CHUNK)], dbuf.at[nb], isem)
            pltpu.async_copy(probs_hbm.at[pl.ds(nc, _CHUNK)], pbuf.at[nb],
                             isem)

            @plsc.parallel_loop(0, groups, 1, unroll=4)
            def group_body(g):
                s16 = sbuf[b, pl.ds(g * _L, _L)]
                d16 = dbuf[b, pl.ds(g * _L, _L)]
                p16 = pbuf[b, pl.ds(g * _L, _L)]
                for c in range(cpt):
                    sf = s16 + (c * n)
                    df = d16 + (c * n)
                    xv = plsc.load_gather(xloc, [sf])
                    plsc.addupdate_scatter(accl, [df], xv * p16)
            return carry
        lax.fori_loop(0, chunks, chunk_body, 0)

        pltpu.make_async_copy(src_hbm.at[pl.ds(0, _CHUNK)],
                              sbuf.at[0], isem).wait()
        pltpu.make_async_copy(dst_hbm.at[pl.ds(0, _CHUNK)],
                              dbuf.at[0], isem).wait()
        pltpu.make_async_copy(probs_hbm.at[pl.ds(0, _CHUNK)],
                              pbuf.at[0], isem).wait()

        pltpu.sync_copy(accl, out_hbm.at[gid])

    return k(xt, srcp, dstp, probsp)


def _combine(x, agg, weight, slw):
    def body(x_ref, a_ref, w_ref, s_ref, o_ref):
        s = s_ref[0, 0]
        o_ref[...] = jnp.clip(
            x_ref[...] * (1.0 + s) + a_ref[...] * w_ref[...], 0.0, 1.0)

    return pl.pallas_call(
        body,
        out_shape=jax.ShapeDtypeStruct(x.shape, x.dtype),
    )(x, agg, weight, slw)


def kernel(x, edge_index, edge_probs, weight, self_loop_weight):
    n, d = x.shape
    e = edge_index.shape[1]
    e_pad = ((e + _CHUNK - 1) // _CHUNK) * _CHUNK
    pad = e_pad - e

    src = jnp.concatenate([edge_index[0], jnp.zeros((pad,), jnp.int32)])
    dst = jnp.concatenate([edge_index[1], jnp.zeros((pad,), jnp.int32)])
    pr = jnp.concatenate(
        [edge_probs.astype(jnp.float32), jnp.zeros((pad,), jnp.float32)])

    cpt = d // _NW
    xt = x.astype(jnp.float32).T.reshape(_NW, cpt * n)

    partials = _sc_scatter(xt, src, dst, pr, n, d)
    agg = partials.reshape(d, n).T

    w2 = weight.astype(jnp.float32).reshape(1, d)
    s2 = jnp.asarray(self_loop_weight, jnp.float32).reshape(1, 1)
    return _combine(x, agg, w2, s2)


# speedup vs baseline: 1.0596x; 1.0596x over previous
import functools

import jax
import jax.numpy as jnp
from jax import lax
from jax.experimental import pallas as pl
from jax.experimental.pallas import tpu as pltpu
from jax.experimental.pallas import tpu_sc as plsc

_NC = 2
_NS = 16
_NW = _NC * _NS
_CHUNK = 4096
_L = 16


def _sc_scatter(xt, srcp, dstp, probsp, n, d):
    e_pad = srcp.shape[0]
    chunks = e_pad // _CHUNK
    cpt = d // _NW
    flat = cpt * n
    groups = _CHUNK // _L

    mesh = plsc.VectorSubcoreMesh(core_axis_name="c", subcore_axis_name="s")

    @functools.partial(
        pl.kernel,
        out_type=jax.ShapeDtypeStruct((_NW, flat), jnp.float32),
        mesh=mesh,
        compiler_params=pltpu.CompilerParams(needs_layout_passes=False),
        scratch_types=[
            pltpu.VMEM((flat,), jnp.float32),
            pltpu.VMEM((flat,), jnp.float32),
            pltpu.VMEM((2, _CHUNK), jnp.int32),
            pltpu.VMEM((2, _CHUNK), jnp.int32),
            pltpu.VMEM((2, _CHUNK), jnp.float32),
            pltpu.SemaphoreType.DMA,
            pltpu.SemaphoreType.DMA,
        ],
    )
    def k(xt_hbm, src_hbm, dst_hbm, probs_hbm, out_hbm,
          xloc, accl, sbuf, dbuf, pbuf, gsem, isem):
        cid = lax.axis_index("c")
        sid = lax.axis_index("s")
        gid = cid * _NS + sid

        pltpu.async_copy(xt_hbm.at[gid], xloc, gsem)

        def zbody(i, carry):
            accl[pl.ds(i * _L, _L)] = jnp.zeros((_L,), jnp.float32)
            return carry
        lax.fori_loop(0, flat // _L, zbody, 0)

        pltpu.async_copy(src_hbm.at[pl.ds(0, _CHUNK)], sbuf.at[0], isem)
        pltpu.async_copy(dst_hbm.at[pl.ds(0, _CHUNK)], dbuf.at[0], isem)
        pltpu.async_copy(probs_hbm.at[pl.ds(0, _CHUNK)], pbuf.at[0], isem)
        pltpu.make_async_copy(xt_hbm.at[gid], xloc, gsem).wait()

        def chunk_body(ci, carry):
            b = ci % 2
            nb = 1 - b
            pltpu.make_async_copy(src_hbm.at[pl.ds(0, _CHUNK)],
                                  sbuf.at[0], isem).wait()
            pltpu.make_async_copy(dst_hbm.at[pl.ds(0, _CHUNK)],
                                  dbuf.at[0], isem).wait()
            pltpu.make_async_copy(probs_hbm.at[pl.ds(0, _CHUNK)],
                                  pbuf.at[0], isem).wait()
            nc = jnp.minimum(ci + 1, chunks - 1) * _CHUNK
            pltpu.async_copy(src_hbm.at[pl.ds(nc, _CHUNK)], sbuf.at[nb], isem)
            pltpu.async_copy(dst_hbm.at[pl.ds(nc, _CHUNK)], dbuf.at[nb], isem)
            pltpu.async_copy(probs_hbm.at[pl.ds(nc, _CHUNK)], pbuf.at[nb],
                             isem)

            @plsc.parallel_loop(0, groups, 1, unroll=4)
            def group_body(g):
                s16 = sbuf[b, pl.ds(g * _L, _L)]
                d16 = dbuf[b, pl.ds(g * _L, _L)]
                p16 = pbuf[b, pl.ds(g * _L, _L)]
                for c in range(cpt):
                    sf = s16 + (c * n)
                    df = d16 + (c * n)
                    xv = plsc.load_gather(xloc, [sf])
                    plsc.addupdate_scatter(accl, [df], xv * p16)
            return carry
        lax.fori_loop(0, chunks, chunk_body, 0)

        pltpu.make_async_copy(src_hbm.at[pl.ds(0, _CHUNK)],
                              sbuf.at[0], isem).wait()
        pltpu.make_async_copy(dst_hbm.at[pl.ds(0, _CHUNK)],
                              dbuf.at[0], isem).wait()
        pltpu.make_async_copy(probs_hbm.at[pl.ds(0, _CHUNK)],
                              pbuf.at[0], isem).wait()

        pltpu.sync_copy(accl, out_hbm.at[gid])

    return k(xt, srcp, dstp, probsp)


def _combine(x, agg, weight, slw):
    def body(x_ref, a_ref, w_ref, s_ref, o_ref):
        s = s_ref[0, 0]
        o_ref[...] = jnp.clip(
            x_ref[...] * (1.0 + s) + a_ref[...] * w_ref[...], 0.0, 1.0)

    return pl.pallas_call(
        body,
        out_shape=jax.ShapeDtypeStruct(x.shape, x.dtype),
    )(x, agg, weight, slw)


def kernel(x, edge_index, edge_probs, weight, self_loop_weight):
    n, d = x.shape
    e = edge_index.shape[1]
    e_pad = ((e + _CHUNK - 1) // _CHUNK) * _CHUNK
    pad = e_pad - e

    src = jnp.concatenate([edge_index[0], jnp.zeros((pad,), jnp.int32)])
    dst = jnp.concatenate([edge_index[1], jnp.zeros((pad,), jnp.int32)])
    pr = jnp.concatenate(
        [edge_probs.astype(jnp.float32), jnp.zeros((pad,), jnp.float32)])

    cpt = d // _NW
    xt = x.astype(jnp.float32).T.reshape(_NW, cpt * n)

    partials = _sc_scatter(xt, src, dst, pr, n, d)
    agg = partials.reshape(d, n).T

    w2 = weight.astype(jnp.float32).reshape(1, d)
    s2 = jnp.asarray(self_loop_weight, jnp.float32).reshape(1, 1)
    return _combine(x, agg, w2, s2)


# speedup vs baseline: 1.1666x; 1.1010x over previous
import functools

import jax
import jax.numpy as jnp
from jax import lax
from jax.experimental import pallas as pl
from jax.experimental.pallas import tpu as pltpu
from jax.experimental.pallas import tpu_sc as plsc

_NC = 2
_NS = 16
_NW = _NC * _NS
_CHUNK = 2048
_L = 16


def _sc_scatter(xt, srcp, dstp, probsp, n, d):
    e_pad = srcp.shape[0]
    chunks = e_pad // _CHUNK
    cpt = d // _NW
    flat = cpt * n
    groups = _CHUNK // _L

    mesh = plsc.VectorSubcoreMesh(core_axis_name="c", subcore_axis_name="s")

    @functools.partial(
        pl.kernel,
        out_type=jax.ShapeDtypeStruct((_NW, flat), jnp.float32),
        mesh=mesh,
        compiler_params=pltpu.CompilerParams(needs_layout_passes=False),
        scratch_types=[
            pltpu.VMEM((flat,), jnp.float32),
            pltpu.VMEM((flat,), jnp.float32),
            pltpu.VMEM((2, _CHUNK), jnp.int32),
            pltpu.VMEM((2, _CHUNK), jnp.int32),
            pltpu.VMEM((2, _CHUNK), jnp.float32),
            pltpu.SemaphoreType.DMA,
            pltpu.SemaphoreType.DMA,
        ],
    )
    def k(xt_hbm, src_hbm, dst_hbm, probs_hbm, out_hbm,
          xloc, accl, sbuf, dbuf, pbuf, gsem, isem):
        cid = lax.axis_index("c")
        sid = lax.axis_index("s")
        gid = cid * _NS + sid

        pltpu.async_copy(xt_hbm.at[gid], xloc, gsem)

        def zbody(i, carry):
            accl[pl.ds(i * _L, _L)] = jnp.zeros((_L,), jnp.float32)
            return carry
        lax.fori_loop(0, flat // _L, zbody, 0)

        pltpu.async_copy(src_hbm.at[pl.ds(0, _CHUNK)], sbuf.at[0], isem)
        pltpu.async_copy(dst_hbm.at[pl.ds(0, _CHUNK)], dbuf.at[0], isem)
        pltpu.async_copy(probs_hbm.at[pl.ds(0, _CHUNK)], pbuf.at[0], isem)
        pltpu.make_async_copy(xt_hbm.at[gid], xloc, gsem).wait()

        def chunk_body(ci, carry):
            b = ci % 2
            nb = 1 - b
            pltpu.make_async_copy(src_hbm.at[pl.ds(0, _CHUNK)],
                                  sbuf.at[0], isem).wait()
            pltpu.make_async_copy(dst_hbm.at[pl.ds(0, _CHUNK)],
                                  dbuf.at[0], isem).wait()
            pltpu.make_async_copy(probs_hbm.at[pl.ds(0, _CHUNK)],
                                  pbuf.at[0], isem).wait()
            nc = jnp.minimum(ci + 1, chunks - 1) * _CHUNK
            pltpu.async_copy(src_hbm.at[pl.ds(nc, _CHUNK)], sbuf.at[nb], isem)
            pltpu.async_copy(dst_hbm.at[pl.ds(nc, _CHUNK)], dbuf.at[nb], isem)
            pltpu.async_copy(probs_hbm.at[pl.ds(nc, _CHUNK)], pbuf.at[nb],
                             isem)

            @plsc.parallel_loop(0, groups, 1, unroll=4)
            def group_body(g):
                s16 = sbuf[b, pl.ds(g * _L, _L)]
                d16 = dbuf[b, pl.ds(g * _L, _L)]
                p16 = pbuf[b, pl.ds(g * _L, _L)]
                for c in range(cpt):
                    sf = s16 + (c * n)
                    df = d16 + (c * n)
                    xv = plsc.load_gather(xloc, [sf])
                    plsc.addupdate_scatter(accl, [df], xv * p16)
            return carry
        lax.fori_loop(0, chunks, chunk_body, 0)

        pltpu.make_async_copy(src_hbm.at[pl.ds(0, _CHUNK)],
                              sbuf.at[0], isem).wait()
        pltpu.make_async_copy(dst_hbm.at[pl.ds(0, _CHUNK)],
                              dbuf.at[0], isem).wait()
        pltpu.make_async_copy(probs_hbm.at[pl.ds(0, _CHUNK)],
                              pbuf.at[0], isem).wait()

        pltpu.sync_copy(accl, out_hbm.at[gid])

    return k(xt, srcp, dstp, probsp)


def _combine(x, agg, weight, slw):
    def body(x_ref, a_ref, w_ref, s_ref, o_ref):
        s = s_ref[0, 0]
        o_ref[...] = jnp.clip(
            x_ref[...] * (1.0 + s) + a_ref[...] * w_ref[...], 0.0, 1.0)

    return pl.pallas_call(
        body,
        out_shape=jax.ShapeDtypeStruct(x.shape, x.dtype),
    )(x, agg, weight, slw)


def kernel(x, edge_index, edge_probs, weight, self_loop_weight):
    n, d = x.shape
    e = edge_index.shape[1]
    e_pad = ((e + _CHUNK - 1) // _CHUNK) * _CHUNK
    pad = e_pad - e

    src = jnp.concatenate([edge_index[0], jnp.zeros((pad,), jnp.int32)])
    dst = jnp.concatenate([edge_index[1], jnp.zeros((pad,), jnp.int32)])
    pr = jnp.concatenate(
        [edge_probs.astype(jnp.float32), jnp.zeros((pad,), jnp.float32)])

    cpt = d // _NW
    xt = x.astype(jnp.float32).T.reshape(_NW, cpt * n)

    partials = _sc_scatter(xt, src, dst, pr, n, d)
    agg = partials.reshape(d, n).T

    w2 = weight.astype(jnp.float32).reshape(1, d)
    s2 = jnp.asarray(self_loop_weight, jnp.float32).reshape(1, 1)
    return _combine(x, agg, w2, s2)
